# baseline (speedup 1.0000x reference)
import functools

import jax
import jax.numpy as jnp
import numpy as np
from jax.experimental import pallas as pl

_MW = 21
_ECHAR = 50
_K = 5
_T = _MW - _K + 1
_EMBED = 128
_VPAD = 128
_EPAD = 64
_KROWS = _MW * _EPAD
_NCOLS = _T * _EMBED


def _build_sel() -> np.ndarray:
    sel = np.full((_KROWS, _T), _ECHAR * _K, dtype=np.int32)
    for t in range(_T):
        for p in range(_MW):
            for e in range(_ECHAR):
                d = 50 * p + e - t
                if d < 0:
                    continue
                k = d % _MW
                c = d // _MW
                if k < _K and c < _ECHAR:
                    sel[_EPAD * p + e, t] = c * _K + k
    return sel


_SEL = _build_sel()

_SELMAT = np.zeros((_KROWS * _T, 256), dtype=np.float32)
_r = np.arange(_KROWS * _T)
_q = _SEL.reshape(-1)
_valid = _q < _ECHAR * _K
_SELMAT[_r[_valid], _q[_valid]] = 1.0


def _word_block(ch, emb2, w2, b2, wp, bp, wg, bg):
    iota = jax.lax.broadcasted_iota(jnp.int32, (1, _VPAD), 1)
    oh = [(ch[:, p:p + 1] == iota).astype(jnp.bfloat16) for p in range(_MW)]
    cols = []
    for p in range(0, _MW - 1, 2):
        ohp = jnp.concatenate([oh[p], oh[p + 1]], axis=1)
        xp = jax.lax.dot_general(ohp, emb2, (((1,), (0,)), ((), ())),
                                 preferred_element_type=jnp.float32)
        cols.append(xp.astype(jnp.bfloat16))
    xl = jax.lax.dot_general(oh[_MW - 1], emb2[:_VPAD, :_EPAD],
                             (((1,), (0,)), ((), ())),
                             preferred_element_type=jnp.float32)
    cols.append(xl.astype(jnp.bfloat16))
    xflat = jnp.concatenate(cols, axis=1)
    convp = jax.lax.dot_general(xflat, w2, (((1,), (0,)), ((), ())),
                                preferred_element_type=jnp.float32)
    m = convp[:, 0:_EMBED]
    for t in range(1, _T):
        m = jnp.maximum(m, convp[:, _EMBED * t:_EMBED * (t + 1)])
    xc = jnp.maximum(m + b2, 0.0)
    xcb = xc.astype(jnp.bfloat16)
    proj = jax.lax.dot_general(xcb, wp, (((1,), (0,)), ((), ())),
                               preferred_element_type=jnp.float32)
    proj = jnp.maximum(proj + bp, 0.0)
    gl = jax.lax.dot_general(xcb, wg, (((1,), (0,)), ((), ())),
                             preferred_element_type=jnp.float32)
    gate = 1.0 / (1.0 + jnp.exp(-(gl + bg)))
    return gate * proj + (1.0 - gate) * xc


def _body(ch_ref, emb_ref, w2_ref, b2_ref, wp_ref, bp_ref, wg_ref, bg_ref,
          out_ref):
    ch = ch_ref[...]
    emb2 = emb_ref[...]
    w2 = w2_ref[...]
    b2, wp, bp, wg, bg = (b2_ref[...], wp_ref[...], bp_ref[...],
                          wg_ref[...], bg_ref[...])
    nb = ch.shape[0]
    h = 256 if nb % 256 == 0 else nb
    for i in range(nb // h):
        out_ref[i * h:(i + 1) * h, :] = _word_block(
            ch[i * h:(i + 1) * h], emb2, w2, b2, wp, bp, wg, bg)


@jax.jit
def kernel(input_tensor, emb_table, conv_w, conv_b, w_proj, b_proj,
           w_gate, b_gate):
    l, b, mw = input_tensor.shape
    n = l * b
    ch = input_tensor.reshape(n, mw).astype(jnp.int32)

    emb_pad = jnp.zeros((_VPAD, _EPAD), jnp.bfloat16)
    emb_pad = emb_pad.at[:96, :_ECHAR].set(emb_table.astype(jnp.bfloat16))
    emb2 = jnp.zeros((2 * _VPAD, 2 * _EPAD), jnp.bfloat16)
    emb2 = emb2.at[:_VPAD, :_EPAD].set(emb_pad)
    emb2 = emb2.at[_VPAD:, _EPAD:].set(emb_pad)
    wmat = conv_w.transpose(1, 2, 0).reshape(_ECHAR * _K, _EMBED)
    wmat_pad = jnp.zeros((256, _EMBED), jnp.bfloat16)
    wmat_pad = wmat_pad.at[:_ECHAR * _K, :].set(wmat.astype(jnp.bfloat16))
    selmat = jnp.asarray(_SELMAT).astype(jnp.bfloat16)
    w2 = jax.lax.dot_general(selmat, wmat_pad, (((1,), (0,)), ((), ())),
                             preferred_element_type=jnp.float32)
    w2 = w2.astype(jnp.bfloat16).reshape(_KROWS, _NCOLS)
    b2 = conv_b.reshape(1, _EMBED)
    wp = w_proj.T.astype(jnp.bfloat16)
    wg = w_gate.T.astype(jnp.bfloat16)
    bp = b_proj.reshape(1, _EMBED)
    bg = b_gate.reshape(1, _EMBED)

    nb = 5120 if n % 5120 == 0 else n
    grid = (n // nb,)
    const = lambda i: (0, 0)
    out = pl.pallas_call(
        _body,
        grid=grid,
        in_specs=[
            pl.BlockSpec((nb, mw), lambda i: (i, 0)),
            pl.BlockSpec((2 * _VPAD, 2 * _EPAD), const),
            pl.BlockSpec((_KROWS, _NCOLS), const),
            pl.BlockSpec((1, _EMBED), const),
            pl.BlockSpec((_EMBED, _EMBED), const),
            pl.BlockSpec((1, _EMBED), const),
            pl.BlockSpec((_EMBED, _EMBED), const),
            pl.BlockSpec((1, _EMBED), const),
        ],
        out_specs=pl.BlockSpec((nb, _EMBED), lambda i: (i, 0)),
        out_shape=jax.ShapeDtypeStruct((n, _EMBED), jnp.float32),
    )(ch, emb2, w2, b2, wp, bp, wg, bg)
    return out.reshape(l, b, _EMBED)


# speedup vs baseline: 1.2026x; 1.2026x over previous
import functools

import jax
import jax.numpy as jnp
import numpy as np
from jax.experimental import pallas as pl

_MW = 21
_ECHAR = 50
_K = 5
_T = _MW - _K + 1
_EMBED = 128
_VPAD = 128
_EPAD = 64
_KROWS = _MW * _EPAD
_NCOLS = _T * _EMBED


def _build_sel() -> np.ndarray:
    sel = np.full((_KROWS, _T), _ECHAR * _K, dtype=np.int32)
    for t in range(_T):
        for p in range(_MW):
            for e in range(_ECHAR):
                d = 50 * p + e - t
                if d < 0:
                    continue
                k = d % _MW
                c = d // _MW
                if k < _K and c < _ECHAR:
                    sel[_EPAD * p + e, t] = c * _K + k
    return sel


_SEL = _build_sel()

_SELMAT = np.zeros((_KROWS * _T, 256), dtype=np.float32)
_r = np.arange(_KROWS * _T)
_q = _SEL.reshape(-1)
_valid = _q < _ECHAR * _K
_SELMAT[_r[_valid], _q[_valid]] = 1.0


def _word_block(ch, emb2, w2, b2, wp, bp, wg, bg):
    iota = jax.lax.broadcasted_iota(jnp.int32, (1, _VPAD), 1)
    oh = [(ch[:, p:p + 1] == iota).astype(jnp.bfloat16) for p in range(_MW)]
    cols = []
    for p in range(0, _MW - 1, 2):
        ohp = jnp.concatenate([oh[p], oh[p + 1]], axis=1)
        xp = jax.lax.dot_general(ohp, emb2, (((1,), (0,)), ((), ())),
                                 preferred_element_type=jnp.float32)
        cols.append(xp.astype(jnp.bfloat16))
    xl = jax.lax.dot_general(oh[_MW - 1], emb2[:_VPAD, :_EPAD],
                             (((1,), (0,)), ((), ())),
                             preferred_element_type=jnp.float32)
    cols.append(xl.astype(jnp.bfloat16))
    xflat = jnp.concatenate(cols, axis=1)
    convp = jax.lax.dot_general(xflat, w2, (((1,), (0,)), ((), ())),
                                preferred_element_type=jnp.float32)
    m = convp[:, 0:_EMBED]
    for t in range(1, _T):
        m = jnp.maximum(m, convp[:, _EMBED * t:_EMBED * (t + 1)])
    xc = jnp.maximum(m + b2, 0.0)
    xcb = xc.astype(jnp.bfloat16)
    proj = jax.lax.dot_general(xcb, wp, (((1,), (0,)), ((), ())),
                               preferred_element_type=jnp.float32)
    proj = jnp.maximum(proj + bp, 0.0)
    gl = jax.lax.dot_general(xcb, wg, (((1,), (0,)), ((), ())),
                             preferred_element_type=jnp.float32)
    gate = 1.0 / (1.0 + jnp.exp(-(gl + bg)))
    return gate * proj + (1.0 - gate) * xc


def _body(ch_ref, emb_ref, w2_ref, b2_ref, wp_ref, bp_ref, wg_ref, bg_ref,
          out_ref):
    ch = ch_ref[...]
    emb2 = emb_ref[...]
    w2 = w2_ref[...]
    b2, wp, bp, wg, bg = (b2_ref[...], wp_ref[...], bp_ref[...],
                          wg_ref[...], bg_ref[...])
    nb = ch.shape[0]
    h = 512 if nb % 512 == 0 else nb
    for i in range(nb // h):
        out_ref[i * h:(i + 1) * h, :] = _word_block(
            ch[i * h:(i + 1) * h], emb2, w2, b2, wp, bp, wg, bg)


@jax.jit
def kernel(input_tensor, emb_table, conv_w, conv_b, w_proj, b_proj,
           w_gate, b_gate):
    l, b, mw = input_tensor.shape
    n = l * b
    ch = input_tensor.reshape(n, mw).astype(jnp.int32)

    emb_pad = jnp.zeros((_VPAD, _EPAD), jnp.bfloat16)
    emb_pad = emb_pad.at[:96, :_ECHAR].set(emb_table.astype(jnp.bfloat16))
    emb2 = jnp.zeros((2 * _VPAD, 2 * _EPAD), jnp.bfloat16)
    emb2 = emb2.at[:_VPAD, :_EPAD].set(emb_pad)
    emb2 = emb2.at[_VPAD:, _EPAD:].set(emb_pad)
    wmat = conv_w.transpose(1, 2, 0).reshape(_ECHAR * _K, _EMBED)
    wmat_pad = jnp.zeros((256, _EMBED), jnp.bfloat16)
    wmat_pad = wmat_pad.at[:_ECHAR * _K, :].set(wmat.astype(jnp.bfloat16))
    selmat = jnp.asarray(_SELMAT).astype(jnp.bfloat16)
    w2 = jax.lax.dot_general(selmat, wmat_pad, (((1,), (0,)), ((), ())),
                             preferred_element_type=jnp.float32)
    w2 = w2.astype(jnp.bfloat16).reshape(_KROWS, _NCOLS)
    b2 = conv_b.reshape(1, _EMBED)
    wp = w_proj.T.astype(jnp.bfloat16)
    wg = w_gate.T.astype(jnp.bfloat16)
    bp = b_proj.reshape(1, _EMBED)
    bg = b_gate.reshape(1, _EMBED)

    nb = 2048 if n % 2048 == 0 else n
    grid = (n // nb,)
    const = lambda i: (0, 0)
    out = pl.pallas_call(
        _body,
        grid=grid,
        in_specs=[
            pl.BlockSpec((nb, mw), lambda i: (i, 0)),
            pl.BlockSpec((2 * _VPAD, 2 * _EPAD), const),
            pl.BlockSpec((_KROWS, _NCOLS), const),
            pl.BlockSpec((1, _EMBED), const),
            pl.BlockSpec((_EMBED, _EMBED), const),
            pl.BlockSpec((1, _EMBED), const),
            pl.BlockSpec((_EMBED, _EMBED), const),
            pl.BlockSpec((1, _EMBED), const),
        ],
        out_specs=pl.BlockSpec((nb, _EMBED), lambda i: (i, 0)),
        out_shape=jax.ShapeDtypeStruct((n, _EMBED), jnp.float32),
    )(ch, emb2, w2, b2, wp, bp, wg, bg)
    return out.reshape(l, b, _EMBED)


# speedup vs baseline: 1.2028x; 1.0002x over previous
import functools

import jax
import jax.numpy as jnp
import numpy as np
from jax.experimental import pallas as pl
from jax.experimental.pallas import tpu as pltpu

_MW = 21
_ECHAR = 50
_K = 5
_T = _MW - _K + 1
_EMBED = 128
_VPAD = 128
_EPAD = 64
_KROWS = _MW * _EPAD
_NCOLS = _T * _EMBED


def _build_sel() -> np.ndarray:
    sel = np.full((_KROWS, _T), _ECHAR * _K, dtype=np.int32)
    for t in range(_T):
        for p in range(_MW):
            for e in range(_ECHAR):
                d = 50 * p + e - t
                if d < 0:
                    continue
                k = d % _MW
                c = d // _MW
                if k < _K and c < _ECHAR:
                    sel[_EPAD * p + e, t] = c * _K + k
    return sel


_SEL = _build_sel()

_SELMAT = np.zeros((_KROWS * _T, 256), dtype=np.float32)
_r = np.arange(_KROWS * _T)
_q = _SEL.reshape(-1)
_valid = _q < _ECHAR * _K
_SELMAT[_r[_valid], _q[_valid]] = 1.0


def _word_block(ch, emb2, w2, b2, wp, bp, wg, bg):
    iota = jax.lax.broadcasted_iota(jnp.int32, (1, _VPAD), 1)
    oh = [(ch[:, p:p + 1] == iota).astype(jnp.bfloat16) for p in range(_MW)]
    cols = []
    for p in range(0, _MW - 1, 2):
        ohp = jnp.concatenate([oh[p], oh[p + 1]], axis=1)
        xp = jax.lax.dot_general(ohp, emb2, (((1,), (0,)), ((), ())),
                                 preferred_element_type=jnp.float32)
        cols.append(xp.astype(jnp.bfloat16))
    xl = jax.lax.dot_general(oh[_MW - 1], emb2[:_VPAD, :_EPAD],
                             (((1,), (0,)), ((), ())),
                             preferred_element_type=jnp.float32)
    cols.append(xl.astype(jnp.bfloat16))
    xflat = jnp.concatenate(cols, axis=1)
    convp = jax.lax.dot_general(xflat, w2, (((1,), (0,)), ((), ())),
                                preferred_element_type=jnp.float32)
    m = convp[:, 0:_EMBED]
    for t in range(1, _T):
        m = jnp.maximum(m, convp[:, _EMBED * t:_EMBED * (t + 1)])
    xc = jnp.maximum(m + b2, 0.0)
    xcb = xc.astype(jnp.bfloat16)
    proj = jax.lax.dot_general(xcb, wp, (((1,), (0,)), ((), ())),
                               preferred_element_type=jnp.float32)
    proj = jnp.maximum(proj + bp, 0.0)
    gl = jax.lax.dot_general(xcb, wg, (((1,), (0,)), ((), ())),
                             preferred_element_type=jnp.float32)
    gate = 1.0 / (1.0 + jnp.exp(-(gl + bg)))
    return gate * proj + (1.0 - gate) * xc


def _body(ch_ref, emb_ref, w2_ref, b2_ref, wp_ref, bp_ref, wg_ref, bg_ref,
          out_ref):
    ch = ch_ref[...]
    emb2 = emb_ref[...]
    w2 = w2_ref[...]
    b2, wp, bp, wg, bg = (b2_ref[...], wp_ref[...], bp_ref[...],
                          wg_ref[...], bg_ref[...])
    nb = ch.shape[0]
    h = 512 if nb % 512 == 0 else nb
    for i in range(nb // h):
        out_ref[i * h:(i + 1) * h, :] = _word_block(
            ch[i * h:(i + 1) * h], emb2, w2, b2, wp, bp, wg, bg)


@jax.jit
def kernel(input_tensor, emb_table, conv_w, conv_b, w_proj, b_proj,
           w_gate, b_gate):
    l, b, mw = input_tensor.shape
    n = l * b
    ch = input_tensor.reshape(n, mw).astype(jnp.int32)

    emb_pad = jnp.zeros((_VPAD, _EPAD), jnp.bfloat16)
    emb_pad = emb_pad.at[:96, :_ECHAR].set(emb_table.astype(jnp.bfloat16))
    emb2 = jnp.zeros((2 * _VPAD, 2 * _EPAD), jnp.bfloat16)
    emb2 = emb2.at[:_VPAD, :_EPAD].set(emb_pad)
    emb2 = emb2.at[_VPAD:, _EPAD:].set(emb_pad)
    wmat = conv_w.transpose(1, 2, 0).reshape(_ECHAR * _K, _EMBED)
    wmat_pad = jnp.zeros((256, _EMBED), jnp.bfloat16)
    wmat_pad = wmat_pad.at[:_ECHAR * _K, :].set(wmat.astype(jnp.bfloat16))
    selmat = jnp.asarray(_SELMAT).astype(jnp.bfloat16)
    w2 = jax.lax.dot_general(selmat, wmat_pad, (((1,), (0,)), ((), ())),
                             preferred_element_type=jnp.float32)
    w2 = w2.astype(jnp.bfloat16).reshape(_KROWS, _NCOLS)
    b2 = conv_b.reshape(1, _EMBED)
    wp = w_proj.T.astype(jnp.bfloat16)
    wg = w_gate.T.astype(jnp.bfloat16)
    bp = b_proj.reshape(1, _EMBED)
    bg = b_gate.reshape(1, _EMBED)

    nb = 2048 if n % 2048 == 0 else n
    grid = (n // nb,)
    const = lambda i: (0, 0)
    out = pl.pallas_call(
        _body,
        grid=grid,
        compiler_params=pltpu.CompilerParams(
            dimension_semantics=("parallel",)),
        in_specs=[
            pl.BlockSpec((nb, mw), lambda i: (i, 0)),
            pl.BlockSpec((2 * _VPAD, 2 * _EPAD), const),
            pl.BlockSpec((_KROWS, _NCOLS), const),
            pl.BlockSpec((1, _EMBED), const),
            pl.BlockSpec((_EMBED, _EMBED), const),
            pl.BlockSpec((1, _EMBED), const),
            pl.BlockSpec((_EMBED, _EMBED), const),
            pl.BlockSpec((1, _EMBED), const),
        ],
        out_specs=pl.BlockSpec((nb, _EMBED), lambda i: (i, 0)),
        out_shape=jax.ShapeDtypeStruct((n, _EMBED), jnp.float32),
    )(ch, emb2, w2, b2, wp, bp, wg, bg)
    return out.reshape(l, b, _EMBED)
